# baseline (speedup 1.0000x reference)
import functools

import jax
import jax.numpy as jnp
from jax import lax
from jax.experimental import pallas as pl
from jax.experimental.pallas import tpu as pltpu
from jax.experimental.pallas import tpu_sc as plsc

S = 2048
D = 768
OUT = 768
E = 8
K = 2
TB = 256
NPAIR = S * K
NBUF = NPAIR + E * TB
NBLK = NBUF // TB
NW = 32


def _gate_body(x_ref, gw_ref, gb_ref, i1_ref, i2_ref, p0_ref, p1_ref):
    x = x_ref[...]
    gw = gw_ref[...]
    logits = lax.dot_general(x, gw, (((1,), (1,)), ((), ())),
                             preferred_element_type=jnp.float32)
    logits = logits + gb_ref[...]
    m = jnp.max(logits, axis=1, keepdims=True)
    ex = jnp.exp(logits - m)
    p = ex / jnp.sum(ex, axis=1, keepdims=True)
    ii = lax.broadcasted_iota(jnp.int32, (S, E), 1)
    m1 = jnp.max(p, axis=1, keepdims=True)
    i1 = jnp.min(jnp.where(p == m1, ii, E), axis=1, keepdims=True)
    pm = jnp.where(ii == i1, -1.0, p)
    m2 = jnp.max(pm, axis=1, keepdims=True)
    i2 = jnp.min(jnp.where(pm == m2, ii, E), axis=1, keepdims=True)
    i1_ref[...] = i1
    i2_ref[...] = i2
    p0_ref[...] = jnp.sum(jnp.where(ii == 0, p, 0.0), axis=1, keepdims=True)
    p1_ref[...] = jnp.sum(jnp.where(ii == 1, p, 0.0), axis=1, keepdims=True)


def _gate(x2d, gate_w, gate_b):
    return pl.pallas_call(
        _gate_body,
        out_shape=(
            jax.ShapeDtypeStruct((S, 1), jnp.int32),
            jax.ShapeDtypeStruct((S, 1), jnp.int32),
            jax.ShapeDtypeStruct((S, 1), jnp.float32),
            jax.ShapeDtypeStruct((S, 1), jnp.float32),
        ),
    )(x2d, gate_w, gate_b.reshape(1, E))


_G_PER_W = NBUF // NW
_G_CH = 64
_G_NCH = _G_PER_W // _G_CH


@functools.cache
def _sc_mesh():
    return plsc.VectorSubcoreMesh(core_axis_name="c", subcore_axis_name="s")


@functools.cache
def _sc_gather_fn():
    @functools.partial(
        pl.kernel,
        out_type=jax.ShapeDtypeStruct((NBUF, D), jnp.float32),
        mesh=_sc_mesh(),
        scratch_types=[
            pltpu.VMEM((_G_PER_W,), jnp.int32),
            pltpu.VMEM((_G_CH, D), jnp.float32),
            pltpu.VMEM((_G_CH, D), jnp.float32),
            pltpu.SemaphoreType.DMA,
            pltpu.SemaphoreType.DMA,
            pltpu.SemaphoreType.DMA,
            pltpu.SemaphoreType.DMA,
        ],
    )
    def gather(x_hbm, idx_hbm, out_hbm, idx_v, buf0, buf1, g0, g1, w0, w1):
        wid = lax.axis_index("s") * 2 + lax.axis_index("c")
        base = wid * _G_PER_W
        pltpu.sync_copy(idx_hbm.at[pl.ds(base, _G_PER_W)], idx_v)
        bufs = (buf0, buf1)
        gsems = (g0, g1)
        wsems = (w0, w1)
        gathers = [None, None]
        writes = [None, None]
        for c in range(_G_NCH):
            b = c % 2
            if writes[b] is not None:
                writes[b].wait()
            gathers[b] = pltpu.async_copy(
                x_hbm.at[idx_v.at[pl.ds(c * _G_CH, _G_CH)]], bufs[b], gsems[b])
            if c >= 1 and gathers[1 - b] is not None:
                gathers[1 - b].wait()
                writes[1 - b] = pltpu.async_copy(
                    bufs[1 - b],
                    out_hbm.at[pl.ds(base + (c - 1) * _G_CH, _G_CH)],
                    wsems[1 - b])
        last = (_G_NCH - 1) % 2
        gathers[last].wait()
        writes[last] = pltpu.async_copy(
            bufs[last], out_hbm.at[pl.ds(base + (_G_NCH - 1) * _G_CH, _G_CH)],
            wsems[last])
        writes[1 - last].wait()
        writes[last].wait()

    return gather


def _sc_gather(x2d, row_token):
    return _sc_gather_fn()(x2d, row_token)


def _mlp_body(be_ref, nv_ref, xg_ref, sc_ref, w1_ref, b1_ref, w2_ref, b2_ref,
              w3_ref, b3_ref, out_ref):
    i = pl.program_id(0)

    @pl.when(i < nv_ref[0])
    def _():
        xb = xg_ref[...]
        h = lax.dot_general(xb, w1_ref[0], (((1,), (1,)), ((), ())),
                            preferred_element_type=jnp.float32)
        h = jnp.maximum(h + b1_ref[0], 0.0)
        h = lax.dot_general(h, w2_ref[0], (((1,), (1,)), ((), ())),
                            preferred_element_type=jnp.float32)
        h = jnp.maximum(h + b2_ref[0], 0.0)
        h = lax.dot_general(h, w3_ref[0], (((1,), (1,)), ((), ())),
                            preferred_element_type=jnp.float32)
        h = h + b3_ref[0]
        out_ref[...] = h * sc_ref[...]


def _mlp(xg, scale, block_expert, nvalid, fc1_w, fc1_b, fc2_w, fc2_b,
         fc3_w, fc3_b):
    grid_spec = pltpu.PrefetchScalarGridSpec(
        num_scalar_prefetch=2,
        grid=(NBLK,),
        in_specs=[
            pl.BlockSpec((TB, D), lambda i, be, nv: (i, 0)),
            pl.BlockSpec((TB, 1), lambda i, be, nv: (i, 0)),
            pl.BlockSpec((1, OUT, D), lambda i, be, nv: (be[i], 0, 0)),
            pl.BlockSpec((1, 1, OUT), lambda i, be, nv: (be[i], 0, 0)),
            pl.BlockSpec((1, OUT, OUT), lambda i, be, nv: (be[i], 0, 0)),
            pl.BlockSpec((1, 1, OUT), lambda i, be, nv: (be[i], 0, 0)),
            pl.BlockSpec((1, OUT, OUT), lambda i, be, nv: (be[i], 0, 0)),
            pl.BlockSpec((1, 1, OUT), lambda i, be, nv: (be[i], 0, 0)),
        ],
        out_specs=pl.BlockSpec((TB, OUT), lambda i, be, nv: (i, 0)),
    )
    return pl.pallas_call(
        _mlp_body,
        grid_spec=grid_spec,
        out_shape=jax.ShapeDtypeStruct((NBUF, OUT), jnp.float32),
    )(block_expert, nvalid, xg, scale, fc1_w, fc1_b.reshape(E, 1, OUT),
      fc2_w, fc2_b.reshape(E, 1, OUT), fc3_w, fc3_b.reshape(E, 1, OUT))


_C_PER_W = S // NW


@functools.cache
def _sc_combine_fn():
    @functools.partial(
        pl.kernel,
        out_type=jax.ShapeDtypeStruct((S, OUT), jnp.float32),
        mesh=_sc_mesh(),
        scratch_types=[
            pltpu.VMEM((_C_PER_W,), jnp.int32),
            pltpu.VMEM((_C_PER_W,), jnp.int32),
            pltpu.VMEM((_C_PER_W, OUT), jnp.float32),
            pltpu.VMEM((_C_PER_W, OUT), jnp.float32),
            pltpu.SemaphoreType.DMA,
            pltpu.SemaphoreType.DMA,
        ],
    )
    def combine(ybuf_hbm, d0_hbm, d1_hbm, out_hbm, i0_v, i1_v, r0_v, r1_v,
                sem0, sem1):
        wid = lax.axis_index("s") * 2 + lax.axis_index("c")
        base = wid * _C_PER_W
        pltpu.sync_copy(d0_hbm.at[pl.ds(base, _C_PER_W)], i0_v)
        pltpu.sync_copy(d1_hbm.at[pl.ds(base, _C_PER_W)], i1_v)
        c0 = pltpu.async_copy(ybuf_hbm.at[i0_v], r0_v, sem0)
        c1 = pltpu.async_copy(ybuf_hbm.at[i1_v], r1_v, sem1)
        c0.wait()
        c1.wait()

        def body(t, carry):
            for j in range(OUT // 16):
                sl = pl.ds(j * 16, 16)
                r0_v[t, sl] = r0_v[t, sl] + r1_v[t, sl]
            return carry

        lax.fori_loop(0, _C_PER_W, body, 0)
        pltpu.sync_copy(r0_v, out_hbm.at[pl.ds(base, _C_PER_W)])

    return combine


def _sc_combine(ybuf, d0, d1):
    return _sc_combine_fn()(ybuf, d0, d1)


def _route(i1, i2, p0, p1):
    ef = jnp.concatenate([i1, i2], axis=1).reshape(-1)
    oh = (ef[:, None] == jnp.arange(E, dtype=jnp.int32)[None, :]).astype(jnp.int32)
    csum = jnp.cumsum(oh, axis=0)
    counts = csum[-1]
    rank = jnp.take_along_axis(csum, ef[:, None], axis=1)[:, 0] - 1
    pc = ((counts + TB - 1) // TB) * TB
    ends = jnp.cumsum(pc)
    starts = ends - pc
    dst = starts[ef] + rank
    row_token = jnp.zeros((NBUF,), jnp.int32).at[dst].set(
        jnp.arange(NPAIR, dtype=jnp.int32) // K)
    pflat = jnp.concatenate([p0, p1], axis=1).reshape(-1)
    scale = jnp.zeros((NBUF, 1), jnp.float32).at[dst, 0].set(pflat)
    nvalid = (ends[-1] // TB).reshape(1).astype(jnp.int32)
    block_expert = jnp.searchsorted(
        ends, jnp.arange(NBLK, dtype=jnp.int32) * TB, side="right")
    block_expert = jnp.minimum(block_expert, E - 1).astype(jnp.int32)
    d0 = dst[0::2]
    d1 = dst[1::2]
    return row_token, scale, block_expert, nvalid, d0, d1


def kernel(x, gate_w, gate_b, fc1_w, fc1_b, fc2_w, fc2_b, fc3_w, fc3_b):
    x2d = x.reshape(S, D)
    i1, i2, p0, p1 = _gate(x2d, gate_w, gate_b)
    row_token, scale, block_expert, nvalid, d0, d1 = _route(i1, i2, p0, p1)
    xg = _sc_gather(x2d, row_token)
    ybuf = _mlp(xg, scale, block_expert, nvalid, fc1_w, fc1_b, fc2_w, fc2_b,
                fc3_w, fc3_b)
    out = _sc_combine(ybuf, d0, d1)
    return out.reshape(1, S, OUT)


# speedup vs baseline: 1.0000x; 1.0000x over previous
import functools

import jax
import jax.numpy as jnp
from jax import lax
from jax.experimental import pallas as pl
from jax.experimental.pallas import tpu as pltpu
from jax.experimental.pallas import tpu_sc as plsc

S = 2048
D = 768
OUT = 768
E = 8
K = 2
TB = 256
NPAIR = S * K
NBUF = NPAIR + E * TB
NBLK = NBUF // TB
NW = 32


def _gate_body(x_ref, gw_ref, gb_ref, i1_ref, i2_ref, p0_ref, p1_ref):
    x = x_ref[...]
    gw = gw_ref[...]
    logits = lax.dot_general(x, gw, (((1,), (1,)), ((), ())),
                             preferred_element_type=jnp.float32)
    logits = logits + gb_ref[...]
    m = jnp.max(logits, axis=1, keepdims=True)
    ex = jnp.exp(logits - m)
    p = ex / jnp.sum(ex, axis=1, keepdims=True)
    ii = lax.broadcasted_iota(jnp.int32, (S, E), 1)
    m1 = jnp.max(p, axis=1, keepdims=True)
    i1 = jnp.min(jnp.where(p == m1, ii, E), axis=1, keepdims=True)
    pm = jnp.where(ii == i1, -1.0, p)
    m2 = jnp.max(pm, axis=1, keepdims=True)
    i2 = jnp.min(jnp.where(pm == m2, ii, E), axis=1, keepdims=True)
    i1_ref[...] = i1
    i2_ref[...] = i2
    p0_ref[...] = jnp.sum(jnp.where(ii == 0, p, 0.0), axis=1, keepdims=True)
    p1_ref[...] = jnp.sum(jnp.where(ii == 1, p, 0.0), axis=1, keepdims=True)


def _gate(x2d, gate_w, gate_b):
    return pl.pallas_call(
        _gate_body,
        out_shape=(
            jax.ShapeDtypeStruct((S, 1), jnp.int32),
            jax.ShapeDtypeStruct((S, 1), jnp.int32),
            jax.ShapeDtypeStruct((S, 1), jnp.float32),
            jax.ShapeDtypeStruct((S, 1), jnp.float32),
        ),
    )(x2d, gate_w, gate_b.reshape(1, E))


_G_PER_W = NBUF // NW
_G_CH = 64
_G_NCH = _G_PER_W // _G_CH


@functools.cache
def _sc_mesh():
    return plsc.VectorSubcoreMesh(core_axis_name="c", subcore_axis_name="s")


@functools.cache
def _sc_gather_fn():
    @functools.partial(
        pl.kernel,
        out_type=jax.ShapeDtypeStruct((NBUF, D), jnp.float32),
        mesh=_sc_mesh(),
        scratch_types=[
            pltpu.VMEM((_G_PER_W,), jnp.int32),
            pltpu.VMEM((_G_CH, D), jnp.float32),
            pltpu.VMEM((_G_CH, D), jnp.float32),
            pltpu.SemaphoreType.DMA,
            pltpu.SemaphoreType.DMA,
            pltpu.SemaphoreType.DMA,
            pltpu.SemaphoreType.DMA,
        ],
    )
    def gather(x_hbm, idx_hbm, out_hbm, idx_v, buf0, buf1, g0, g1, w0, w1):
        wid = lax.axis_index("s") * 2 + lax.axis_index("c")
        base = wid * _G_PER_W
        pltpu.sync_copy(idx_hbm.at[pl.ds(base, _G_PER_W)], idx_v)
        bufs = (buf0, buf1)
        gsems = (g0, g1)
        wsems = (w0, w1)
        gathers = [None, None]
        writes = [None, None]
        for c in range(_G_NCH):
            b = c % 2
            if writes[b] is not None:
                writes[b].wait()
            gathers[b] = pltpu.async_copy(
                x_hbm.at[idx_v.at[pl.ds(c * _G_CH, _G_CH)]], bufs[b], gsems[b])
            if c >= 1 and gathers[1 - b] is not None:
                gathers[1 - b].wait()
                writes[1 - b] = pltpu.async_copy(
                    bufs[1 - b],
                    out_hbm.at[pl.ds(base + (c - 1) * _G_CH, _G_CH)],
                    wsems[1 - b])
        last = (_G_NCH - 1) % 2
        gathers[last].wait()
        writes[last] = pltpu.async_copy(
            bufs[last], out_hbm.at[pl.ds(base + (_G_NCH - 1) * _G_CH, _G_CH)],
            wsems[last])
        writes[1 - last].wait()
        writes[last].wait()

    return gather


def _sc_gather(x2d, row_token):
    return _sc_gather_fn()(x2d, row_token)


def _mlp_body(be_ref, nv_ref, xg_ref, sc_ref, w1_ref, b1_ref, w2_ref, b2_ref,
              w3_ref, b3_ref, out_ref):
    i = pl.program_id(0)

    @pl.when(i < nv_ref[0])
    def _():
        xb = xg_ref[...]
        h = lax.dot_general(xb, w1_ref[0], (((1,), (1,)), ((), ())),
                            preferred_element_type=jnp.float32)
        h = jnp.maximum(h + b1_ref[0], 0.0)
        h = lax.dot_general(h, w2_ref[0], (((1,), (1,)), ((), ())),
                            preferred_element_type=jnp.float32)
        h = jnp.maximum(h + b2_ref[0], 0.0)
        h = lax.dot_general(h, w3_ref[0], (((1,), (1,)), ((), ())),
                            preferred_element_type=jnp.float32)
        h = h + b3_ref[0]
        out_ref[...] = h * sc_ref[...]


def _mlp(xg, scale, block_expert, nvalid, fc1_w, fc1_b, fc2_w, fc2_b,
         fc3_w, fc3_b):
    grid_spec = pltpu.PrefetchScalarGridSpec(
        num_scalar_prefetch=2,
        grid=(NBLK,),
        in_specs=[
            pl.BlockSpec((TB, D), lambda i, be, nv: (i, 0)),
            pl.BlockSpec((TB, 1), lambda i, be, nv: (i, 0)),
            pl.BlockSpec((1, OUT, D), lambda i, be, nv: (be[i], 0, 0)),
            pl.BlockSpec((1, 1, OUT), lambda i, be, nv: (be[i], 0, 0)),
            pl.BlockSpec((1, OUT, OUT), lambda i, be, nv: (be[i], 0, 0)),
            pl.BlockSpec((1, 1, OUT), lambda i, be, nv: (be[i], 0, 0)),
            pl.BlockSpec((1, OUT, OUT), lambda i, be, nv: (be[i], 0, 0)),
            pl.BlockSpec((1, 1, OUT), lambda i, be, nv: (be[i], 0, 0)),
        ],
        out_specs=pl.BlockSpec((TB, OUT), lambda i, be, nv: (i, 0)),
    )
    return pl.pallas_call(
        _mlp_body,
        grid_spec=grid_spec,
        out_shape=jax.ShapeDtypeStruct((NBUF, OUT), jnp.float32),
    )(block_expert, nvalid, xg, scale, fc1_w, fc1_b.reshape(E, 1, OUT),
      fc2_w, fc2_b.reshape(E, 1, OUT), fc3_w, fc3_b.reshape(E, 1, OUT))


_C_PER_W = S // NW


@functools.cache
def _sc_combine_fn():
    @functools.partial(
        pl.kernel,
        out_type=jax.ShapeDtypeStruct((S, OUT), jnp.float32),
        mesh=_sc_mesh(),
        scratch_types=[
            pltpu.VMEM((_C_PER_W,), jnp.int32),
            pltpu.VMEM((_C_PER_W,), jnp.int32),
            pltpu.VMEM((_C_PER_W, OUT), jnp.float32),
            pltpu.VMEM((_C_PER_W, OUT), jnp.float32),
            pltpu.SemaphoreType.DMA,
            pltpu.SemaphoreType.DMA,
        ],
    )
    def combine(ybuf_hbm, d0_hbm, d1_hbm, out_hbm, i0_v, i1_v, r0_v, r1_v,
                sem0, sem1):
        wid = lax.axis_index("s") * 2 + lax.axis_index("c")
        base = wid * _C_PER_W
        pltpu.sync_copy(d0_hbm.at[pl.ds(base, _C_PER_W)], i0_v)
        pltpu.sync_copy(d1_hbm.at[pl.ds(base, _C_PER_W)], i1_v)
        c0 = pltpu.async_copy(ybuf_hbm.at[i0_v], r0_v, sem0)
        c1 = pltpu.async_copy(ybuf_hbm.at[i1_v], r1_v, sem1)
        c0.wait()
        c1.wait()

        @plsc.parallel_loop(0, _C_PER_W, 1, unroll=2)
        def _(t):
            for j in range(OUT // 16):
                sl = pl.ds(j * 16, 16)
                plsc.addupdate(r0_v.at[t, sl], r1_v[t, sl])
        pltpu.sync_copy(r0_v, out_hbm.at[pl.ds(base, _C_PER_W)])

    return combine


def _sc_combine(ybuf, d0, d1):
    return _sc_combine_fn()(ybuf, d0, d1)


def _route(i1, i2, p0, p1):
    ef = jnp.concatenate([i1, i2], axis=1).reshape(-1)
    oh = (ef[:, None] == jnp.arange(E, dtype=jnp.int32)[None, :]).astype(jnp.int32)
    csum = jnp.cumsum(oh, axis=0)
    counts = csum[-1]
    rank = jnp.take_along_axis(csum, ef[:, None], axis=1)[:, 0] - 1
    pc = ((counts + TB - 1) // TB) * TB
    ends = jnp.cumsum(pc)
    starts = ends - pc
    dst = starts[ef] + rank
    row_token = jnp.zeros((NBUF,), jnp.int32).at[dst].set(
        jnp.arange(NPAIR, dtype=jnp.int32) // K)
    pflat = jnp.concatenate([p0, p1], axis=1).reshape(-1)
    scale = jnp.zeros((NBUF, 1), jnp.float32).at[dst, 0].set(pflat)
    nvalid = (ends[-1] // TB).reshape(1).astype(jnp.int32)
    block_expert = jnp.searchsorted(
        ends, jnp.arange(NBLK, dtype=jnp.int32) * TB, side="right")
    block_expert = jnp.minimum(block_expert, E - 1).astype(jnp.int32)
    d0 = dst[0::2]
    d1 = dst[1::2]
    return row_token, scale, block_expert, nvalid, d0, d1


def kernel(x, gate_w, gate_b, fc1_w, fc1_b, fc2_w, fc2_b, fc3_w, fc3_b):
    x2d = x.reshape(S, D)
    i1, i2, p0, p1 = _gate(x2d, gate_w, gate_b)
    row_token, scale, block_expert, nvalid, d0, d1 = _route(i1, i2, p0, p1)
    xg = _sc_gather(x2d, row_token)
    ybuf = _mlp(xg, scale, block_expert, nvalid, fc1_w, fc1_b, fc2_w, fc2_b,
                fc3_w, fc3_b)
    out = _sc_combine(ybuf, d0, d1)
    return out.reshape(1, S, OUT)


# speedup vs baseline: 1.5588x; 1.5588x over previous
import functools

import jax
import jax.numpy as jnp
from jax import lax
from jax.experimental import pallas as pl
from jax.experimental.pallas import tpu as pltpu
from jax.experimental.pallas import tpu_sc as plsc

S = 2048
D = 768
OUT = 768
E = 8
K = 2
TB = 256
NPAIR = S * K
NBUF = NPAIR + E * TB
NBLK = NBUF // TB
NW = 32


def _gate_body(x_ref, gw_ref, gb_ref, i1_ref, i2_ref, p0_ref, p1_ref):
    x = x_ref[...]
    gw = gw_ref[...]
    logits = lax.dot_general(x, gw, (((1,), (1,)), ((), ())),
                             preferred_element_type=jnp.float32)
    logits = logits + gb_ref[...]
    m = jnp.max(logits, axis=1, keepdims=True)
    ex = jnp.exp(logits - m)
    p = ex / jnp.sum(ex, axis=1, keepdims=True)
    ii = lax.broadcasted_iota(jnp.int32, (S, E), 1)
    m1 = jnp.max(p, axis=1, keepdims=True)
    i1 = jnp.min(jnp.where(p == m1, ii, E), axis=1, keepdims=True)
    pm = jnp.where(ii == i1, -1.0, p)
    m2 = jnp.max(pm, axis=1, keepdims=True)
    i2 = jnp.min(jnp.where(pm == m2, ii, E), axis=1, keepdims=True)
    i1_ref[...] = i1
    i2_ref[...] = i2
    p0_ref[...] = jnp.sum(jnp.where(ii == 0, p, 0.0), axis=1, keepdims=True)
    p1_ref[...] = jnp.sum(jnp.where(ii == 1, p, 0.0), axis=1, keepdims=True)


def _gate(x2d, gate_w, gate_b):
    return pl.pallas_call(
        _gate_body,
        out_shape=(
            jax.ShapeDtypeStruct((S, 1), jnp.int32),
            jax.ShapeDtypeStruct((S, 1), jnp.int32),
            jax.ShapeDtypeStruct((S, 1), jnp.float32),
            jax.ShapeDtypeStruct((S, 1), jnp.float32),
        ),
    )(x2d, gate_w, gate_b.reshape(1, E))


_G_PER_W = NBUF // NW
_G_CH = 64
_G_NCH = _G_PER_W // _G_CH


@functools.cache
def _sc_mesh():
    return plsc.VectorSubcoreMesh(core_axis_name="c", subcore_axis_name="s")


@functools.cache
def _sc_gather_fn():
    @functools.partial(
        pl.kernel,
        out_type=jax.ShapeDtypeStruct((NBUF, D), jnp.float32),
        mesh=_sc_mesh(),
        scratch_types=[
            pltpu.VMEM((_G_PER_W,), jnp.int32),
            pltpu.VMEM((_G_CH, D), jnp.float32),
            pltpu.VMEM((_G_CH, D), jnp.float32),
            pltpu.SemaphoreType.DMA,
            pltpu.SemaphoreType.DMA,
            pltpu.SemaphoreType.DMA,
            pltpu.SemaphoreType.DMA,
        ],
    )
    def gather(x_hbm, idx_hbm, out_hbm, idx_v, buf0, buf1, g0, g1, w0, w1):
        wid = lax.axis_index("s") * 2 + lax.axis_index("c")
        base = wid * _G_PER_W
        pltpu.sync_copy(idx_hbm.at[pl.ds(base, _G_PER_W)], idx_v)
        bufs = (buf0, buf1)
        gsems = (g0, g1)
        wsems = (w0, w1)
        gathers = [None, None]
        writes = [None, None]
        for c in range(_G_NCH):
            b = c % 2
            if writes[b] is not None:
                writes[b].wait()
            gathers[b] = pltpu.async_copy(
                x_hbm.at[idx_v.at[pl.ds(c * _G_CH, _G_CH)]], bufs[b], gsems[b])
            if c >= 1 and gathers[1 - b] is not None:
                gathers[1 - b].wait()
                writes[1 - b] = pltpu.async_copy(
                    bufs[1 - b],
                    out_hbm.at[pl.ds(base + (c - 1) * _G_CH, _G_CH)],
                    wsems[1 - b])
        last = (_G_NCH - 1) % 2
        gathers[last].wait()
        writes[last] = pltpu.async_copy(
            bufs[last], out_hbm.at[pl.ds(base + (_G_NCH - 1) * _G_CH, _G_CH)],
            wsems[last])
        writes[1 - last].wait()
        writes[last].wait()

    return gather


def _sc_gather(x2d, row_token):
    return _sc_gather_fn()(x2d, row_token)


def _mlp_body(be_ref, nv_ref, xg_ref, sc_ref, w1_ref, b1_ref, w2_ref, b2_ref,
              w3_ref, b3_ref, out_ref):
    i = pl.program_id(0)

    @pl.when(i < nv_ref[0])
    def _():
        xb = xg_ref[...]
        h = lax.dot_general(xb, w1_ref[0], (((1,), (1,)), ((), ())),
                            preferred_element_type=jnp.float32)
        h = jnp.maximum(h + b1_ref[0], 0.0)
        h = lax.dot_general(h, w2_ref[0], (((1,), (1,)), ((), ())),
                            preferred_element_type=jnp.float32)
        h = jnp.maximum(h + b2_ref[0], 0.0)
        h = lax.dot_general(h, w3_ref[0], (((1,), (1,)), ((), ())),
                            preferred_element_type=jnp.float32)
        h = h + b3_ref[0]
        out_ref[...] = h * sc_ref[...]


def _mlp(xg, scale, block_expert, nvalid, fc1_w, fc1_b, fc2_w, fc2_b,
         fc3_w, fc3_b):
    grid_spec = pltpu.PrefetchScalarGridSpec(
        num_scalar_prefetch=2,
        grid=(NBLK,),
        in_specs=[
            pl.BlockSpec((TB, D), lambda i, be, nv: (i, 0)),
            pl.BlockSpec((TB, 1), lambda i, be, nv: (i, 0)),
            pl.BlockSpec((1, OUT, D), lambda i, be, nv: (be[i], 0, 0)),
            pl.BlockSpec((1, 1, OUT), lambda i, be, nv: (be[i], 0, 0)),
            pl.BlockSpec((1, OUT, OUT), lambda i, be, nv: (be[i], 0, 0)),
            pl.BlockSpec((1, 1, OUT), lambda i, be, nv: (be[i], 0, 0)),
            pl.BlockSpec((1, OUT, OUT), lambda i, be, nv: (be[i], 0, 0)),
            pl.BlockSpec((1, 1, OUT), lambda i, be, nv: (be[i], 0, 0)),
        ],
        out_specs=pl.BlockSpec((TB, OUT), lambda i, be, nv: (i, 0)),
    )
    return pl.pallas_call(
        _mlp_body,
        grid_spec=grid_spec,
        out_shape=jax.ShapeDtypeStruct((NBUF, OUT), jnp.float32),
    )(block_expert, nvalid, xg, scale, fc1_w, fc1_b.reshape(E, 1, OUT),
      fc2_w, fc2_b.reshape(E, 1, OUT), fc3_w, fc3_b.reshape(E, 1, OUT))


_C_PER_W = S // NW


@functools.cache
def _sc_combine_fn():
    @functools.partial(
        pl.kernel,
        out_type=jax.ShapeDtypeStruct((S, OUT), jnp.float32),
        mesh=_sc_mesh(),
        scratch_types=[
            pltpu.VMEM((_C_PER_W,), jnp.int32),
            pltpu.VMEM((_C_PER_W,), jnp.int32),
            pltpu.VMEM((_C_PER_W, OUT), jnp.float32),
            pltpu.VMEM((_C_PER_W, OUT), jnp.float32),
            pltpu.SemaphoreType.DMA,
            pltpu.SemaphoreType.DMA,
        ],
    )
    def combine(ybuf_hbm, d0_hbm, d1_hbm, out_hbm, i0_v, i1_v, r0_v, r1_v,
                sem0, sem1):
        wid = lax.axis_index("s") * 2 + lax.axis_index("c")
        base = wid * _C_PER_W
        pltpu.sync_copy(d0_hbm.at[pl.ds(base, _C_PER_W)], i0_v)
        pltpu.sync_copy(d1_hbm.at[pl.ds(base, _C_PER_W)], i1_v)
        c0 = pltpu.async_copy(ybuf_hbm.at[i0_v], r0_v, sem0)
        c1 = pltpu.async_copy(ybuf_hbm.at[i1_v], r1_v, sem1)
        c0.wait()
        c1.wait()

        @plsc.parallel_loop(0, _C_PER_W, 1, unroll=2)
        def _(t):
            for j in range(OUT // 16):
                sl = pl.ds(j * 16, 16)
                plsc.addupdate(r0_v.at[t, sl], r1_v[t, sl])
        pltpu.sync_copy(r0_v, out_hbm.at[pl.ds(base, _C_PER_W)])

    return combine


def _sc_combine(ybuf, d0, d1):
    return _sc_combine_fn()(ybuf, d0, d1)


def _route(i1, i2, p0, p1):
    ef = jnp.concatenate([i1, i2], axis=1).reshape(-1)
    oh = (ef[:, None] == jnp.arange(E, dtype=jnp.int32)[None, :]).astype(jnp.int32)
    csum = jnp.cumsum(oh, axis=0)
    counts = csum[-1]
    rank = jnp.take_along_axis(csum, ef[:, None], axis=1)[:, 0] - 1
    pc = ((counts + TB - 1) // TB) * TB
    ends = jnp.cumsum(pc)
    starts = ends - pc
    dst = starts[ef] + rank
    row_token = (jnp.arange(NBUF, dtype=jnp.int32) % S).at[dst].set(
        jnp.arange(NPAIR, dtype=jnp.int32) // K)
    pflat = jnp.concatenate([p0, p1], axis=1).reshape(-1)
    scale = jnp.zeros((NBUF, 1), jnp.float32).at[dst, 0].set(pflat)
    nvalid = (ends[-1] // TB).reshape(1).astype(jnp.int32)
    block_expert = jnp.searchsorted(
        ends, jnp.arange(NBLK, dtype=jnp.int32) * TB, side="right")
    block_expert = jnp.minimum(block_expert, E - 1).astype(jnp.int32)
    d0 = dst[0::2]
    d1 = dst[1::2]
    return row_token, scale, block_expert, nvalid, d0, d1


def kernel(x, gate_w, gate_b, fc1_w, fc1_b, fc2_w, fc2_b, fc3_w, fc3_b):
    x2d = x.reshape(S, D)
    i1, i2, p0, p1 = _gate(x2d, gate_w, gate_b)
    row_token, scale, block_expert, nvalid, d0, d1 = _route(i1, i2, p0, p1)
    xg = _sc_gather(x2d, row_token)
    ybuf = _mlp(xg, scale, block_expert, nvalid, fc1_w, fc1_b, fc2_w, fc2_b,
                fc3_w, fc3_b)
    out = _sc_combine(ybuf, d0, d1)
    return out.reshape(1, S, OUT)


# speedup vs baseline: 2.2812x; 1.4634x over previous
import functools

import jax
import jax.numpy as jnp
from jax import lax
from jax.experimental import pallas as pl
from jax.experimental.pallas import tpu as pltpu
from jax.experimental.pallas import tpu_sc as plsc

S = 2048
D = 768
OUT = 768
E = 8
K = 2
TB = 256
NPAIR = S * K
NBUF = NPAIR + E * TB
NBLK = NBUF // TB
NW = 32
CB = 256


def _gate_body(x_ref, gw_ref, gb_ref, d0_ref, d1_ref, p0_ref, p1_ref,
               be_ref, nv_ref):
    x = x_ref[...]
    gw = gw_ref[...]
    logits = lax.dot_general(x, gw, (((1,), (1,)), ((), ())),
                             preferred_element_type=jnp.float32)
    logits = logits + gb_ref[...]
    m = jnp.max(logits, axis=1, keepdims=True)
    ex = jnp.exp(logits - m)
    p = ex / jnp.sum(ex, axis=1, keepdims=True)
    ii = lax.broadcasted_iota(jnp.int32, (S, E), 1)
    m1 = jnp.max(p, axis=1, keepdims=True)
    i1 = jnp.min(jnp.where(p == m1, ii, E), axis=1, keepdims=True)
    pm = jnp.where(ii == i1, -1.0, p)
    m2 = jnp.max(pm, axis=1, keepdims=True)
    i2 = jnp.min(jnp.where(pm == m2, ii, E), axis=1, keepdims=True)
    p0_ref[...] = jnp.sum(jnp.where(ii == 0, p, 0.0), axis=1, keepdims=True)
    p1_ref[...] = jnp.sum(jnp.where(ii == 1, p, 0.0), axis=1, keepdims=True)

    oh1 = (ii == i1).astype(jnp.float32)
    oh2 = (ii == i2).astype(jnp.float32)
    oh = oh1 + oh2
    ri = lax.broadcasted_iota(jnp.int32, (CB, CB), 0)
    ci = lax.broadcasted_iota(jnp.int32, (CB, CB), 1)
    tril = (ri >= ci).astype(jnp.float32)
    blocks = []
    carry = jnp.zeros((1, E), jnp.float32)
    for c in range(S // CB):
        blk = oh[c * CB:(c + 1) * CB, :]
        cum = lax.dot_general(tril, blk, (((1,), (0,)), ((), ())),
                              preferred_element_type=jnp.float32) + carry
        blocks.append(cum)
        carry = cum[CB - 1:CB, :]
    ct = jnp.concatenate(blocks, axis=0)

    counts = ct[S - 1:S, :]
    pc = jnp.floor((counts + (TB - 1)) * (1.0 / TB)) * TB
    ii8 = lax.broadcasted_iota(jnp.int32, (E, E), 0)
    jj8 = lax.broadcasted_iota(jnp.int32, (E, E), 1)
    cummat = (ii8 <= jj8).astype(jnp.float32)
    ends = lax.dot_general(pc, cummat, (((1,), (0,)), ((), ())),
                           preferred_element_type=jnp.float32)
    starts = ends - pc

    slot = ct + starts - 1.0
    d0 = jnp.sum(jnp.where(ii == i1, slot, 0.0), axis=1, keepdims=True)
    d1 = jnp.sum(jnp.where(ii == i2, slot, 0.0), axis=1, keepdims=True)
    d0_ref[...] = d0.astype(jnp.int32)
    d1_ref[...] = d1.astype(jnp.int32)

    bi = lax.broadcasted_iota(jnp.int32, (32, E), 0).astype(jnp.float32) * float(TB)
    be = jnp.sum((ends <= bi).astype(jnp.int32), axis=1, keepdims=True)
    be_ref[...] = jnp.minimum(be, E - 1)
    jje = lax.broadcasted_iota(jnp.int32, (1, E), 1)
    total = jnp.sum(jnp.where(jje == E - 1, ends, 0.0), axis=1, keepdims=True)
    nv_ref[...] = (total * (1.0 / TB)).astype(jnp.int32)


def _gate(x2d, gate_w, gate_b):
    return pl.pallas_call(
        _gate_body,
        out_shape=(
            jax.ShapeDtypeStruct((S, 1), jnp.int32),
            jax.ShapeDtypeStruct((S, 1), jnp.int32),
            jax.ShapeDtypeStruct((S, 1), jnp.float32),
            jax.ShapeDtypeStruct((S, 1), jnp.float32),
            jax.ShapeDtypeStruct((32, 1), jnp.int32),
            jax.ShapeDtypeStruct((1, 1), jnp.int32),
        ),
    )(x2d, gate_w, gate_b.reshape(1, E))


_X_PER_W = S // NW


@functools.cache
def _sc_mesh():
    return plsc.VectorSubcoreMesh(core_axis_name="c", subcore_axis_name="s")


@functools.cache
def _sc_scatter_fn():
    @functools.partial(
        pl.kernel,
        out_type=jax.ShapeDtypeStruct((NBUF, D), jnp.float32),
        mesh=_sc_mesh(),
        scratch_types=[
            pltpu.VMEM((_X_PER_W, D), jnp.float32),
            pltpu.VMEM((_X_PER_W,), jnp.int32),
            pltpu.VMEM((_X_PER_W,), jnp.int32),
            pltpu.SemaphoreType.DMA,
            pltpu.SemaphoreType.DMA,
        ],
    )
    def scatter(x_hbm, d0_hbm, d1_hbm, out_hbm, xrows_v, i0_v, i1_v,
                sem0, sem1):
        wid = lax.axis_index("s") * 2 + lax.axis_index("c")
        base = wid * _X_PER_W
        pltpu.sync_copy(x_hbm.at[pl.ds(base, _X_PER_W)], xrows_v)
        pltpu.sync_copy(d0_hbm.at[pl.ds(base, _X_PER_W)], i0_v)
        pltpu.sync_copy(d1_hbm.at[pl.ds(base, _X_PER_W)], i1_v)
        c0 = pltpu.async_copy(xrows_v, out_hbm.at[i0_v], sem0)
        c1 = pltpu.async_copy(xrows_v, out_hbm.at[i1_v], sem1)
        c0.wait()
        c1.wait()

    return scatter


def _sc_scatter(x2d, d0, d1):
    return _sc_scatter_fn()(x2d, d0, d1)


def _mlp_body(be_ref, nv_ref, xg_ref, w1_ref, b1_ref, w2_ref, b2_ref,
              w3_ref, b3_ref, out_ref):
    i = pl.program_id(0)

    @pl.when(i < nv_ref[0])
    def _():
        xb = xg_ref[...]
        h = lax.dot_general(xb, w1_ref[0], (((1,), (1,)), ((), ())),
                            preferred_element_type=jnp.float32)
        h = jnp.maximum(h + b1_ref[0], 0.0)
        h = lax.dot_general(h, w2_ref[0], (((1,), (1,)), ((), ())),
                            preferred_element_type=jnp.float32)
        h = jnp.maximum(h + b2_ref[0], 0.0)
        h = lax.dot_general(h, w3_ref[0], (((1,), (1,)), ((), ())),
                            preferred_element_type=jnp.float32)
        out_ref[...] = h + b3_ref[0]


def _mlp(xg, block_expert, nvalid, fc1_w, fc1_b, fc2_w, fc2_b, fc3_w, fc3_b):
    grid_spec = pltpu.PrefetchScalarGridSpec(
        num_scalar_prefetch=2,
        grid=(NBLK,),
        in_specs=[
            pl.BlockSpec((TB, D), lambda i, be, nv: (i, 0)),
            pl.BlockSpec((1, OUT, D), lambda i, be, nv: (be[i], 0, 0)),
            pl.BlockSpec((1, 1, OUT), lambda i, be, nv: (be[i], 0, 0)),
            pl.BlockSpec((1, OUT, OUT), lambda i, be, nv: (be[i], 0, 0)),
            pl.BlockSpec((1, 1, OUT), lambda i, be, nv: (be[i], 0, 0)),
            pl.BlockSpec((1, OUT, OUT), lambda i, be, nv: (be[i], 0, 0)),
            pl.BlockSpec((1, 1, OUT), lambda i, be, nv: (be[i], 0, 0)),
        ],
        out_specs=pl.BlockSpec((TB, OUT), lambda i, be, nv: (i, 0)),
    )
    return pl.pallas_call(
        _mlp_body,
        grid_spec=grid_spec,
        out_shape=jax.ShapeDtypeStruct((NBUF, OUT), jnp.float32),
    )(block_expert, nvalid, xg, fc1_w, fc1_b.reshape(E, 1, OUT),
      fc2_w, fc2_b.reshape(E, 1, OUT), fc3_w, fc3_b.reshape(E, 1, OUT))


_C_PER_W = S // NW
_NL = 16


def _lane_bcast(v, l):
    idx = jnp.full((_NL,), l, jnp.int32)
    return lax.gather(
        v, idx[:, None],
        lax.GatherDimensionNumbers(offset_dims=(), collapsed_slice_dims=(0,),
                                   start_index_map=(0,)),
        (1,), mode=lax.GatherScatterMode.PROMISE_IN_BOUNDS)


@functools.cache
def _sc_combine_fn():
    @functools.partial(
        pl.kernel,
        out_type=jax.ShapeDtypeStruct((S, OUT), jnp.float32),
        mesh=_sc_mesh(),
        scratch_types=[
            pltpu.VMEM((_C_PER_W,), jnp.int32),
            pltpu.VMEM((_C_PER_W,), jnp.int32),
            pltpu.VMEM((_C_PER_W,), jnp.float32),
            pltpu.VMEM((_C_PER_W,), jnp.float32),
            pltpu.VMEM((_C_PER_W, OUT), jnp.float32),
            pltpu.VMEM((_C_PER_W, OUT), jnp.float32),
            pltpu.SemaphoreType.DMA,
            pltpu.SemaphoreType.DMA,
        ],
    )
    def combine(ybuf_hbm, d0_hbm, d1_hbm, p0_hbm, p1_hbm, out_hbm,
                i0_v, i1_v, p0_v, p1_v, r0_v, r1_v, sem0, sem1):
        wid = lax.axis_index("s") * 2 + lax.axis_index("c")
        base = wid * _C_PER_W
        pltpu.sync_copy(d0_hbm.at[pl.ds(base, _C_PER_W)], i0_v)
        pltpu.sync_copy(d1_hbm.at[pl.ds(base, _C_PER_W)], i1_v)
        c0 = pltpu.async_copy(ybuf_hbm.at[i0_v], r0_v, sem0)
        c1 = pltpu.async_copy(ybuf_hbm.at[i1_v], r1_v, sem1)
        pltpu.sync_copy(p0_hbm.at[pl.ds(base, _C_PER_W)], p0_v)
        pltpu.sync_copy(p1_hbm.at[pl.ds(base, _C_PER_W)], p1_v)
        c0.wait()
        c1.wait()

        @plsc.parallel_loop(0, _C_PER_W, 1, unroll=2)
        def _(t):
            cbase = (t // _NL) * _NL
            lane = t - cbase
            b0 = _lane_bcast(p0_v[pl.ds(cbase, _NL)], lane)
            b1 = _lane_bcast(p1_v[pl.ds(cbase, _NL)], lane)
            for j in range(OUT // _NL):
                sl = pl.ds(j * _NL, _NL)
                r0_v[t, sl] = r0_v[t, sl] * b0 + r1_v[t, sl] * b1
        pltpu.sync_copy(r0_v, out_hbm.at[pl.ds(base, _C_PER_W)])

    return combine


def _sc_combine(ybuf, d0, d1, p0, p1):
    return _sc_combine_fn()(ybuf, d0, d1, p0, p1)


def kernel(x, gate_w, gate_b, fc1_w, fc1_b, fc2_w, fc2_b, fc3_w, fc3_b):
    x2d = x.reshape(S, D)
    d0, d1, p0, p1, be, nv = _gate(x2d, gate_w, gate_b)
    d0 = d0.reshape(S)
    d1 = d1.reshape(S)
    block_expert = be.reshape(32)[:NBLK]
    nvalid = nv.reshape(1)
    xg = _sc_scatter(x2d, d0, d1)
    ybuf = _mlp(xg, block_expert, nvalid, fc1_w, fc1_b, fc2_w, fc2_b,
                fc3_w, fc3_b)
    out = _sc_combine(ybuf, d0, d1, p0.reshape(S), p1.reshape(S))
    return out.reshape(1, S, OUT)


# speedup vs baseline: 2.4532x; 1.0754x over previous
import functools

import jax
import jax.numpy as jnp
from jax import lax
from jax.experimental import pallas as pl
from jax.experimental.pallas import tpu as pltpu
from jax.experimental.pallas import tpu_sc as plsc

S = 2048
D = 768
OUT = 768
E = 8
K = 2
TB = 256
NPAIR = S * K
NBUF = NPAIR + E * TB
NBLK = NBUF // TB
NW = 32
CB = 256


def _gate_body(x_ref, gw_ref, gb_ref, d0_ref, d1_ref, p0_ref, p1_ref,
               be_ref):
    x = x_ref[...]
    gw = gw_ref[...]
    logits = lax.dot_general(gw, x, (((1,), (1,)), ((), ())),
                             preferred_element_type=jnp.float32)
    logits = logits + gb_ref[...]
    m = jnp.max(logits, axis=0, keepdims=True)
    ex = jnp.exp(logits - m)
    p = ex / jnp.sum(ex, axis=0, keepdims=True)
    ii = lax.broadcasted_iota(jnp.int32, (E, S), 0)
    m1 = jnp.max(p, axis=0, keepdims=True)
    i1 = jnp.min(jnp.where(p == m1, ii, E), axis=0, keepdims=True)
    pm = jnp.where(ii == i1, -1.0, p)
    m2 = jnp.max(pm, axis=0, keepdims=True)
    i2 = jnp.min(jnp.where(pm == m2, ii, E), axis=0, keepdims=True)
    p0_ref[...] = jnp.sum(jnp.where(ii == 0, p, 0.0), axis=0)
    p1_ref[...] = jnp.sum(jnp.where(ii == 1, p, 0.0), axis=0)

    oh = (ii == i1).astype(jnp.float32) + (ii == i2).astype(jnp.float32)
    ri = lax.broadcasted_iota(jnp.int32, (CB, CB), 0)
    ci = lax.broadcasted_iota(jnp.int32, (CB, CB), 1)
    ut = (ri <= ci).astype(jnp.float32)
    blocks = []
    carry = jnp.zeros((E, 1), jnp.float32)
    for c in range(S // CB):
        blk = oh[:, c * CB:(c + 1) * CB]
        cum = lax.dot_general(blk, ut, (((1,), (0,)), ((), ())),
                              preferred_element_type=jnp.float32) + carry
        blocks.append(cum)
        carry = cum[:, CB - 1:CB]
    ct = jnp.concatenate(blocks, axis=1)

    counts = ct[:, S - 1:S]
    pc = jnp.floor((counts + (TB - 1)) * (1.0 / TB)) * TB
    ii8 = lax.broadcasted_iota(jnp.int32, (E, E), 0)
    jj8 = lax.broadcasted_iota(jnp.int32, (E, E), 1)
    cummat = (jj8 <= ii8).astype(jnp.float32)
    ends = lax.dot_general(cummat, pc, (((1,), (0,)), ((), ())),
                           preferred_element_type=jnp.float32)
    starts = ends - pc

    slot = ct + starts - 1.0
    d0 = jnp.sum(jnp.where(ii == i1, slot, 0.0), axis=0)
    d1 = jnp.sum(jnp.where(ii == i2, slot, 0.0), axis=0)
    d0_ref[...] = d0.astype(jnp.int32)
    d1_ref[...] = d1.astype(jnp.int32)

    bi = lax.broadcasted_iota(jnp.int32, (E, 32), 1).astype(jnp.float32) * float(TB)
    be = jnp.sum((ends <= bi).astype(jnp.int32), axis=0)
    be = jnp.minimum(be, E - 1)
    jj32 = lax.broadcasted_iota(jnp.int32, (E, 32), 1)
    ii32 = lax.broadcasted_iota(jnp.int32, (E, 32), 0)
    total = jnp.sum(jnp.where((jj32 == 31) & (ii32 == E - 1),
                              ends * (1.0 / TB), 0.0), axis=0).astype(jnp.int32)
    be_ref[...] = jnp.where(jnp.arange(32) == 31, total, be)


def _gate(x2d, gate_w, gate_b):
    return pl.pallas_call(
        _gate_body,
        out_shape=(
            jax.ShapeDtypeStruct((S,), jnp.int32),
            jax.ShapeDtypeStruct((S,), jnp.int32),
            jax.ShapeDtypeStruct((S,), jnp.float32),
            jax.ShapeDtypeStruct((S,), jnp.float32),
            jax.ShapeDtypeStruct((32,), jnp.int32),
        ),
    )(x2d, gate_w, gate_b.reshape(E, 1))


_X_PER_W = S // NW


@functools.cache
def _sc_mesh():
    return plsc.VectorSubcoreMesh(core_axis_name="c", subcore_axis_name="s")


@functools.cache
def _sc_scatter_fn():
    @functools.partial(
        pl.kernel,
        out_type=jax.ShapeDtypeStruct((NBUF, D), jnp.float32),
        mesh=_sc_mesh(),
        scratch_types=[
            pltpu.VMEM((_X_PER_W, D), jnp.float32),
            pltpu.VMEM((_X_PER_W,), jnp.int32),
            pltpu.VMEM((_X_PER_W,), jnp.int32),
            pltpu.SemaphoreType.DMA,
            pltpu.SemaphoreType.DMA,
        ],
    )
    def scatter(x_hbm, d0_hbm, d1_hbm, out_hbm, xrows_v, i0_v, i1_v,
                sem0, sem1):
        wid = lax.axis_index("s") * 2 + lax.axis_index("c")
        base = wid * _X_PER_W
        pltpu.sync_copy(x_hbm.at[pl.ds(base, _X_PER_W)], xrows_v)
        pltpu.sync_copy(d0_hbm.at[pl.ds(base, _X_PER_W)], i0_v)
        pltpu.sync_copy(d1_hbm.at[pl.ds(base, _X_PER_W)], i1_v)
        c0 = pltpu.async_copy(xrows_v, out_hbm.at[i0_v], sem0)
        c1 = pltpu.async_copy(xrows_v, out_hbm.at[i1_v], sem1)
        c0.wait()
        c1.wait()

    return scatter


def _sc_scatter(x2d, d0, d1):
    return _sc_scatter_fn()(x2d, d0, d1)


def _mlp_body(be_ref, xg_ref, w1_ref, b1_ref, w2_ref, b2_ref,
              w3_ref, b3_ref, out_ref):
    i = pl.program_id(0)

    @pl.when(i < be_ref[31])
    def _():
        xb = xg_ref[...]
        h = lax.dot_general(xb, w1_ref[0], (((1,), (1,)), ((), ())),
                            preferred_element_type=jnp.float32)
        h = jnp.maximum(h + b1_ref[0], 0.0)
        h = lax.dot_general(h, w2_ref[0], (((1,), (1,)), ((), ())),
                            preferred_element_type=jnp.float32)
        h = jnp.maximum(h + b2_ref[0], 0.0)
        h = lax.dot_general(h, w3_ref[0], (((1,), (1,)), ((), ())),
                            preferred_element_type=jnp.float32)
        out_ref[...] = h + b3_ref[0]


def _mlp(xg, benv, fc1_w, fc1_b, fc2_w, fc2_b, fc3_w, fc3_b):
    grid_spec = pltpu.PrefetchScalarGridSpec(
        num_scalar_prefetch=1,
        grid=(NBLK,),
        in_specs=[
            pl.BlockSpec((TB, D), lambda i, be: (i, 0)),
            pl.BlockSpec((1, OUT, D), lambda i, be: (be[i], 0, 0)),
            pl.BlockSpec((1, 1, OUT), lambda i, be: (be[i], 0, 0)),
            pl.BlockSpec((1, OUT, OUT), lambda i, be: (be[i], 0, 0)),
            pl.BlockSpec((1, 1, OUT), lambda i, be: (be[i], 0, 0)),
            pl.BlockSpec((1, OUT, OUT), lambda i, be: (be[i], 0, 0)),
            pl.BlockSpec((1, 1, OUT), lambda i, be: (be[i], 0, 0)),
        ],
        out_specs=pl.BlockSpec((TB, OUT), lambda i, be: (i, 0)),
    )
    return pl.pallas_call(
        _mlp_body,
        grid_spec=grid_spec,
        out_shape=jax.ShapeDtypeStruct((NBUF, OUT), jnp.float32),
    )(benv, xg, fc1_w, fc1_b.reshape(E, 1, OUT),
      fc2_w, fc2_b.reshape(E, 1, OUT), fc3_w, fc3_b.reshape(E, 1, OUT))


_C_PER_W = S // NW
_NL = 16


def _lane_bcast(v, l):
    idx = jnp.full((_NL,), l, jnp.int32)
    return lax.gather(
        v, idx[:, None],
        lax.GatherDimensionNumbers(offset_dims=(), collapsed_slice_dims=(0,),
                                   start_index_map=(0,)),
        (1,), mode=lax.GatherScatterMode.PROMISE_IN_BOUNDS)


@functools.cache
def _sc_combine_fn():
    @functools.partial(
        pl.kernel,
        out_type=jax.ShapeDtypeStruct((S, OUT), jnp.float32),
        mesh=_sc_mesh(),
        scratch_types=[
            pltpu.VMEM((_C_PER_W,), jnp.int32),
            pltpu.VMEM((_C_PER_W,), jnp.int32),
            pltpu.VMEM((_C_PER_W,), jnp.float32),
            pltpu.VMEM((_C_PER_W,), jnp.float32),
            pltpu.VMEM((_C_PER_W, OUT), jnp.float32),
            pltpu.VMEM((_C_PER_W, OUT), jnp.float32),
            pltpu.SemaphoreType.DMA,
            pltpu.SemaphoreType.DMA,
        ],
    )
    def combine(ybuf_hbm, d0_hbm, d1_hbm, p0_hbm, p1_hbm, out_hbm,
                i0_v, i1_v, p0_v, p1_v, r0_v, r1_v, sem0, sem1):
        wid = lax.axis_index("s") * 2 + lax.axis_index("c")
        base = wid * _C_PER_W
        pltpu.sync_copy(d0_hbm.at[pl.ds(base, _C_PER_W)], i0_v)
        pltpu.sync_copy(d1_hbm.at[pl.ds(base, _C_PER_W)], i1_v)
        c0 = pltpu.async_copy(ybuf_hbm.at[i0_v], r0_v, sem0)
        c1 = pltpu.async_copy(ybuf_hbm.at[i1_v], r1_v, sem1)
        pltpu.sync_copy(p0_hbm.at[pl.ds(base, _C_PER_W)], p0_v)
        pltpu.sync_copy(p1_hbm.at[pl.ds(base, _C_PER_W)], p1_v)
        c0.wait()
        c1.wait()

        @plsc.parallel_loop(0, _C_PER_W, 1, unroll=2)
        def _(t):
            cbase = (t // _NL) * _NL
            lane = t - cbase
            b0 = _lane_bcast(p0_v[pl.ds(cbase, _NL)], lane)
            b1 = _lane_bcast(p1_v[pl.ds(cbase, _NL)], lane)
            for j in range(OUT // _NL):
                sl = pl.ds(j * _NL, _NL)
                r0_v[t, sl] = r0_v[t, sl] * b0 + r1_v[t, sl] * b1
        pltpu.sync_copy(r0_v, out_hbm.at[pl.ds(base, _C_PER_W)])

    return combine


def _sc_combine(ybuf, d0, d1, p0, p1):
    return _sc_combine_fn()(ybuf, d0, d1, p0, p1)


def kernel(x, gate_w, gate_b, fc1_w, fc1_b, fc2_w, fc2_b, fc3_w, fc3_b):
    x2d = x.reshape(S, D)
    d0, d1, p0, p1, benv = _gate(x2d, gate_w, gate_b)
    xg = _sc_scatter(x2d, d0, d1)
    ybuf = _mlp(xg, benv, fc1_w, fc1_b, fc2_w, fc2_b, fc3_w, fc3_b)
    out = _sc_combine(ybuf, d0, d1, p0, p1)
    return out.reshape(1, S, OUT)
